# baseline (speedup 1.0000x reference)
import functools

import jax
import jax.numpy as jnp
from jax import lax
from jax.experimental import pallas as pl
from jax.experimental.pallas import tpu as pltpu
from jax.experimental.pallas import tpu_sc as plsc

B = 4096
CTX = 20
D = 128
V = 100000

NC = 2
NS = 16
NW = NC * NS
ROWS_PER_W = B // NW
CHUNK_ROWS = 16
N_CHUNKS = ROWS_PER_W // CHUNK_ROWS
IDX_PER_CHUNK = CHUNK_ROWS * CTX
GATHER = 80
G_PER_CHUNK = IDX_PER_CHUNK // GATHER


def _sc_pool(idx_flat, emb_table):
    mesh = plsc.VectorSubcoreMesh(core_axis_name="c", subcore_axis_name="s")

    @functools.partial(
        pl.kernel,
        mesh=mesh,
        out_type=jax.ShapeDtypeStruct((B, D), jnp.float32),
        scratch_types=[
            pltpu.VMEM((ROWS_PER_W * CTX,), jnp.int32),
            pltpu.VMEM((IDX_PER_CHUNK, D), jnp.float32),
            pltpu.VMEM((IDX_PER_CHUNK, D), jnp.float32),
            pltpu.VMEM((CHUNK_ROWS, D), jnp.float32),
            pltpu.SemaphoreType.DMA,
            pltpu.SemaphoreType.DMA,
        ],
    )
    def pool_kernel(idx_hbm, table_hbm, out_hbm, idx_v, rows0, rows1, outb,
                    sem0, sem1):
        wid = lax.axis_index("s") * NC + lax.axis_index("c")
        base_row = wid * ROWS_PER_W
        pltpu.sync_copy(idx_hbm.at[pl.ds(base_row * CTX, ROWS_PER_W * CTX)],
                        idx_v)

        bufs = (rows0, rows1)
        sems = (sem0, sem1)

        def issue(c):
            buf, sem = bufs[c % 2], sems[c % 2]
            cps = []
            for g in range(G_PER_CHUNK):
                off = c * IDX_PER_CHUNK + g * GATHER
                cps.append(pltpu.async_copy(
                    table_hbm.at[idx_v.at[pl.ds(off, GATHER)]],
                    buf.at[pl.ds(g * GATHER, GATHER)],
                    sem))
            return cps

        def reduce_chunk(c):
            buf = bufs[c % 2]

            @pl.loop(0, CHUNK_ROWS)
            def _(j):
                rbase = j * CTX
                for grp in range(D // 16):
                    sl = pl.ds(grp * 16, 16)
                    acc = buf[rbase, sl]
                    for t in range(1, CTX):
                        acc = acc + buf[rbase + t, sl]
                    outb[j, sl] = acc * (1.0 / CTX)

            pltpu.sync_copy(
                outb, out_hbm.at[pl.ds(base_row + c * CHUNK_ROWS, CHUNK_ROWS)])

        pending = issue(0)
        for c in range(N_CHUNKS):
            nxt = issue(c + 1) if c + 1 < N_CHUNKS else []
            for cp in pending:
                cp.wait()
            reduce_chunk(c)
            pending = nxt

    return pool_kernel(idx_flat, emb_table)


BN = 1024
GN = -(-V // BN)


def _mm_body(pooled_ref, w_ref, b_ref, out_ref):
    a = pooled_ref[...].astype(jnp.bfloat16)
    w = w_ref[...].astype(jnp.bfloat16)
    acc = lax.dot_general(a, w, (((1,), (1,)), ((), ())),
                          preferred_element_type=jnp.float32)
    out_ref[...] = acc + b_ref[...]


def _tc_matmul(pooled, fc_w, fc_b):
    return pl.pallas_call(
        _mm_body,
        grid=(GN,),
        in_specs=[
            pl.BlockSpec((B, D), lambda n: (0, 0)),
            pl.BlockSpec((BN, D), lambda n: (n, 0)),
            pl.BlockSpec((1, BN), lambda n: (0, n)),
        ],
        out_specs=pl.BlockSpec((B, BN), lambda n: (0, n)),
        out_shape=jax.ShapeDtypeStruct((B, V), jnp.float32),
    )(pooled, fc_w, fc_b.reshape(1, V))


def kernel(inputs, emb_table, fc_w, fc_b):
    idx_flat = inputs.reshape(-1)
    pooled = _sc_pool(idx_flat, emb_table)
    return _tc_matmul(pooled, fc_w, fc_b)


# speedup vs baseline: 3.4835x; 3.4835x over previous
import functools

import jax
import jax.numpy as jnp
from jax import lax
from jax.experimental import pallas as pl
from jax.experimental.pallas import tpu as pltpu
from jax.experimental.pallas import tpu_sc as plsc

B = 4096
CTX = 20
D = 128
V = 100000

NC = 2
NS = 16
NW = NC * NS
ROWS_PER_W = B // NW
CHUNK_ROWS = 16
N_CHUNKS = ROWS_PER_W // CHUNK_ROWS
IDX_PER_CHUNK = CHUNK_ROWS * CTX
GATHER = 80
G_PER_CHUNK = IDX_PER_CHUNK // GATHER


def _sc_pool(idx_flat, emb_table):
    mesh = plsc.VectorSubcoreMesh(core_axis_name="c", subcore_axis_name="s")

    @functools.partial(
        pl.kernel,
        mesh=mesh,
        out_type=jax.ShapeDtypeStruct((B, D), jnp.float32),
        scratch_types=[
            pltpu.VMEM((ROWS_PER_W * CTX,), jnp.int32),
            pltpu.VMEM((IDX_PER_CHUNK, D), jnp.float32),
            pltpu.VMEM((IDX_PER_CHUNK, D), jnp.float32),
            pltpu.VMEM((CHUNK_ROWS, D), jnp.float32),
            pltpu.SemaphoreType.DMA,
            pltpu.SemaphoreType.DMA,
        ],
    )
    def pool_kernel(idx_hbm, table_hbm, out_hbm, idx_v, rows0, rows1, outb,
                    sem0, sem1):
        wid = lax.axis_index("s") * NC + lax.axis_index("c")
        base_row = wid * ROWS_PER_W
        pltpu.sync_copy(idx_hbm.at[pl.ds(base_row * CTX, ROWS_PER_W * CTX)],
                        idx_v)

        bufs = (rows0, rows1)
        sems = (sem0, sem1)

        def issue(c):
            buf, sem = bufs[c % 2], sems[c % 2]
            cps = []
            for g in range(G_PER_CHUNK):
                off = c * IDX_PER_CHUNK + g * GATHER
                cps.append(pltpu.async_copy(
                    table_hbm.at[idx_v.at[pl.ds(off, GATHER)]],
                    buf.at[pl.ds(g * GATHER, GATHER)],
                    sem))
            return cps

        def reduce_chunk(c):
            buf = bufs[c % 2]

            @pl.loop(0, CHUNK_ROWS)
            def _(j):
                rbase = j * CTX
                for grp in range(D // 16):
                    sl = pl.ds(grp * 16, 16)
                    acc = buf[rbase, sl]
                    for t in range(1, CTX):
                        acc = acc + buf[rbase + t, sl]
                    outb[j, sl] = acc * (1.0 / CTX)

            pltpu.sync_copy(
                outb, out_hbm.at[pl.ds(base_row + c * CHUNK_ROWS, CHUNK_ROWS)])

        pending = issue(0)
        for c in range(N_CHUNKS):
            nxt = issue(c + 1) if c + 1 < N_CHUNKS else []
            for cp in pending:
                cp.wait()
            reduce_chunk(c)
            pending = nxt

    return pool_kernel(idx_flat, emb_table)


BV = 1024
GV = -(-V // BV)


def _mm_body(w_ref, pooled_ref, b_ref, out_ref):
    w = w_ref[...].astype(jnp.bfloat16)
    a = pooled_ref[...].astype(jnp.bfloat16)
    acc = lax.dot_general(w, a, (((1,), (1,)), ((), ())),
                          preferred_element_type=jnp.float32)
    ones_row = jnp.ones((1, B), dtype=jnp.float32)
    bias_outer = lax.dot_general(b_ref[...], ones_row, (((0,), (0,)), ((), ())),
                                 preferred_element_type=jnp.float32)
    out_ref[...] = acc + bias_outer


def _tc_matmul(pooled, fc_w, fc_b):
    outT = pl.pallas_call(
        _mm_body,
        grid=(GV,),
        in_specs=[
            pl.BlockSpec((BV, D), lambda n: (n, 0)),
            pl.BlockSpec((B, D), lambda n: (0, 0)),
            pl.BlockSpec((1, BV), lambda n: (0, n)),
        ],
        out_specs=pl.BlockSpec((BV, B), lambda n: (n, 0)),
        out_shape=jax.ShapeDtypeStruct((V, B), jnp.float32),
    )(fc_w, pooled, fc_b.reshape(1, V))
    return outT.T


def kernel(inputs, emb_table, fc_w, fc_b):
    idx_flat = inputs.reshape(-1)
    pooled = _sc_pool(idx_flat, emb_table)
    return _tc_matmul(pooled, fc_w, fc_b)
